# Initial kernel scaffold; baseline (speedup 1.0000x reference)
import jax
import jax.numpy as jnp
from jax.experimental import pallas as pl


def kernel(plugin_ids, ctx_seq, past_action_ids, emb_table, W_ih, W_hh, b_ih, b_hh):
    raise NotImplementedError("write your pallas kernel here")


# speedup vs baseline: 2.0835x; 2.0835x over previous
import functools

import jax
import jax.numpy as jnp
from jax import lax
from jax.experimental import pallas as pl
from jax.experimental.pallas import tpu as pltpu
from jax.experimental.pallas import tpu_sc as plsc


def _sc_embed(plugin_ids, past_ids, emb_table, num_cores, num_subcores):
    B, = plugin_ids.shape
    HIST = past_ids.shape[1]
    D = emb_table.shape[1]
    NW = num_cores * num_subcores
    BPW = B // NW

    mesh = plsc.VectorSubcoreMesh(
        core_axis_name="c", subcore_axis_name="s",
        num_cores=num_cores, num_subcores=num_subcores)

    @functools.partial(
        pl.kernel,
        mesh=mesh,
        compiler_params=pltpu.CompilerParams(use_tc_tiling_on_sc=False),
        out_type=(
            jax.ShapeDtypeStruct((B, D), jnp.float32),
            jax.ShapeDtypeStruct((B, D), jnp.float32),
        ),
        scratch_types=[
            pltpu.VMEM((BPW,), jnp.int32),
            pltpu.VMEM((BPW, D), jnp.float32),
            pltpu.VMEM((BPW, HIST), jnp.int32),
            pltpu.VMEM((2, HIST, D), jnp.float32),
            pltpu.VMEM((BPW, D), jnp.float32),
            pltpu.SemaphoreType.DMA,
            pltpu.SemaphoreType.DMA,
        ],
    )
    def k(plug_hbm, past_hbm, table_hbm, plug_out, mean_out,
          pidx_v, prow_v, hidx_v, rows_v, acc_v, sem0, sem1):
        w = lax.axis_index("s") * num_cores + lax.axis_index("c")
        base = w * BPW

        pltpu.sync_copy(plug_hbm.at[pl.ds(base, BPW)], pidx_v)
        pltpu.async_copy(table_hbm.at[pidx_v], prow_v, sem0).wait()
        pltpu.sync_copy(prow_v, plug_out.at[pl.ds(base, BPW)])

        pltpu.sync_copy(past_hbm.at[pl.ds(base, BPW)], hidx_v)

        sems = (sem0, sem1)
        inv = jnp.float32(1.0 / HIST)
        zero = jnp.zeros((16,), jnp.float32)

        pltpu.async_copy(table_hbm.at[hidx_v.at[0]], rows_v.at[0], sem0)

        def outer(i, carry):
            e0 = i * 2
            for b in (0, 1):
                e = e0 + b
                nxt = e + 1

                @pl.when(nxt < BPW)
                def _issue():
                    pltpu.async_copy(table_hbm.at[hidx_v.at[nxt]],
                                     rows_v.at[1 - b], sems[1 - b])

                pltpu.make_async_copy(table_hbm.at[hidx_v.at[0]],
                                      rows_v.at[b], sems[b]).wait()

                def red(j, acc):
                    a0, a1 = acc
                    return (a0 + rows_v[b, j, 0:16],
                            a1 + rows_v[b, j, 16:32])

                a0, a1 = lax.fori_loop(0, HIST, red, (zero, zero))
                acc_v[e, 0:16] = a0 * inv
                acc_v[e, 16:32] = a1 * inv
            return carry

        lax.fori_loop(0, BPW // 2, outer, 0)
        pltpu.sync_copy(acc_v, mean_out.at[pl.ds(base, BPW)])

    return k(plugin_ids, past_ids, emb_table)


def _gru(ctx_t, W_ihT, W_hhT, b_ih2, b_hh2):
    STEPS, B, H = ctx_t.shape
    G = 3 * H

    def body(x_ref, wih_ref, whh_ref, bih_ref, bhh_ref, out_ref):
        t = pl.program_id(0)

        @pl.when(t == 0)
        def _init():
            out_ref[...] = jnp.zeros_like(out_ref)

        h = out_ref[...]
        xt = x_ref[0]
        gi = jnp.dot(xt, wih_ref[...], preferred_element_type=jnp.float32)
        gi = gi + bih_ref[...]
        gh = jnp.dot(h, whh_ref[...], preferred_element_type=jnp.float32)
        gh = gh + bhh_ref[...]
        r = jax.nn.sigmoid(gi[:, 0:H] + gh[:, 0:H])
        z = jax.nn.sigmoid(gi[:, H:2 * H] + gh[:, H:2 * H])
        n = jnp.tanh(gi[:, 2 * H:] + r * gh[:, 2 * H:])
        out_ref[...] = (1.0 - z) * n + z * h

    return pl.pallas_call(
        body,
        grid=(STEPS,),
        in_specs=[
            pl.BlockSpec((1, B, H), lambda t: (t, 0, 0)),
            pl.BlockSpec((H, G), lambda t: (0, 0)),
            pl.BlockSpec((H, G), lambda t: (0, 0)),
            pl.BlockSpec((1, G), lambda t: (0, 0)),
            pl.BlockSpec((1, G), lambda t: (0, 0)),
        ],
        out_specs=pl.BlockSpec((B, H), lambda t: (0, 0)),
        out_shape=jax.ShapeDtypeStruct((B, H), jnp.float32),
    )(ctx_t, W_ihT, W_hhT, b_ih2, b_hh2)


def kernel(plugin_ids, ctx_seq, past_action_ids, emb_table, W_ih, W_hh, b_ih, b_hh):
    info = plsc.get_sparse_core_info()
    plug = plugin_ids.astype(jnp.int32)
    past = past_action_ids.astype(jnp.int32)
    plug_emb, past_mean = _sc_embed(plug, past, emb_table,
                                    info.num_cores, info.num_subcores)
    ctx_t = jnp.swapaxes(ctx_seq, 0, 1)
    h_ctx = _gru(ctx_t, W_ih.T, W_hh.T,
                 b_ih.reshape(1, -1), b_hh.reshape(1, -1))
    return jnp.concatenate([plug_emb, h_ctx, past_mean], axis=-1)


# speedup vs baseline: 2.2778x; 1.0933x over previous
import functools

import jax
import jax.numpy as jnp
from jax import lax
from jax.experimental import pallas as pl
from jax.experimental.pallas import tpu as pltpu
from jax.experimental.pallas import tpu_sc as plsc


def _sc_embed(plugin_ids, past_ids, emb_table, num_cores, num_subcores):
    B, = plugin_ids.shape
    HIST = past_ids.shape[1]
    D = emb_table.shape[1]
    NW = num_cores * num_subcores
    BPW = B // NW

    mesh = plsc.VectorSubcoreMesh(
        core_axis_name="c", subcore_axis_name="s",
        num_cores=num_cores, num_subcores=num_subcores)

    @functools.partial(
        pl.kernel,
        mesh=mesh,
        compiler_params=pltpu.CompilerParams(use_tc_tiling_on_sc=False),
        out_type=(
            jax.ShapeDtypeStruct((B, D), jnp.float32),
            jax.ShapeDtypeStruct((B, D), jnp.float32),
        ),
        scratch_types=[
            pltpu.VMEM((BPW,), jnp.int32),
            pltpu.VMEM((BPW, D), jnp.float32),
            pltpu.VMEM((BPW, HIST), jnp.int32),
            pltpu.VMEM((2, HIST, D), jnp.float32),
            pltpu.VMEM((BPW, D), jnp.float32),
            pltpu.SemaphoreType.DMA,
            pltpu.SemaphoreType.DMA,
        ],
    )
    def k(plug_hbm, past_hbm, table_hbm, plug_out, mean_out,
          pidx_v, prow_v, hidx_v, rows_v, acc_v, sem0, sem1):
        w = lax.axis_index("s") * num_cores + lax.axis_index("c")
        base = w * BPW

        pltpu.sync_copy(plug_hbm.at[pl.ds(base, BPW)], pidx_v)
        pltpu.async_copy(table_hbm.at[pidx_v], prow_v, sem0).wait()
        pltpu.sync_copy(prow_v, plug_out.at[pl.ds(base, BPW)])

        pltpu.sync_copy(past_hbm.at[pl.ds(base, BPW)], hidx_v)

        sems = (sem0, sem1)
        inv = jnp.float32(1.0 / HIST)
        zero = jnp.zeros((16,), jnp.float32)

        pltpu.async_copy(table_hbm.at[hidx_v.at[0]], rows_v.at[0], sem0)

        def outer(i, carry):
            e0 = i * 2
            for b in (0, 1):
                e = e0 + b
                nxt = e + 1

                @pl.when(nxt < BPW)
                def _issue():
                    pltpu.async_copy(table_hbm.at[hidx_v.at[nxt]],
                                     rows_v.at[1 - b], sems[1 - b])

                pltpu.make_async_copy(table_hbm.at[hidx_v.at[0]],
                                      rows_v.at[b], sems[b]).wait()

                acc = [zero] * 8
                for j in range(HIST):
                    p = j % 4
                    acc[p] = acc[p] + rows_v[b, j, 0:16]
                    acc[4 + p] = acc[4 + p] + rows_v[b, j, 16:32]
                a0 = (acc[0] + acc[1]) + (acc[2] + acc[3])
                a1 = (acc[4] + acc[5]) + (acc[6] + acc[7])
                acc_v[e, 0:16] = a0 * inv
                acc_v[e, 16:32] = a1 * inv
            return carry

        lax.fori_loop(0, BPW // 2, outer, 0)
        pltpu.sync_copy(acc_v, mean_out.at[pl.ds(base, BPW)])

    return k(plugin_ids, past_ids, emb_table)


_GRU_CHUNK = 512


def _gru(ctx2d, W_ih, W_hh, b_ih2, b_hh2, steps, h_dim):
    B = ctx2d.shape[0]
    H = h_dim
    G = 3 * H
    C = _GRU_CHUNK
    NCH = B // C

    def body(x_ref, wih_ref, whh_ref, bih_ref, bhh_ref, out_ref):
        wih = wih_ref[...]
        whh = whh_ref[...]
        bih = bih_ref[...]
        bhh = bhh_ref[...]
        h = jnp.zeros((H, C), jnp.float32)
        dn_t = (((1,), (1,)), ((), ()))
        dn_n = (((1,), (0,)), ((), ()))
        for t in range(steps):
            xt = x_ref[:, t * H:(t + 1) * H]
            gi = lax.dot_general(wih, xt, dn_t,
                                 preferred_element_type=jnp.float32) + bih
            gh = lax.dot_general(whh, h, dn_n,
                                 preferred_element_type=jnp.float32) + bhh
            r = jax.nn.sigmoid(gi[0:H] + gh[0:H])
            z = jax.nn.sigmoid(gi[H:2 * H] + gh[H:2 * H])
            n = jnp.tanh(gi[2 * H:] + r * gh[2 * H:])
            h = (1.0 - z) * n + z * h
        out_ref[...] = h

    return pl.pallas_call(
        body,
        grid=(NCH,),
        in_specs=[
            pl.BlockSpec((C, steps * H), lambda i: (i, 0)),
            pl.BlockSpec((G, H), lambda i: (0, 0)),
            pl.BlockSpec((G, H), lambda i: (0, 0)),
            pl.BlockSpec((G, 1), lambda i: (0, 0)),
            pl.BlockSpec((G, 1), lambda i: (0, 0)),
        ],
        out_specs=pl.BlockSpec((H, C), lambda i: (0, i)),
        out_shape=jax.ShapeDtypeStruct((H, B), jnp.float32),
    )(ctx2d, W_ih, W_hh, b_ih2, b_hh2)


def kernel(plugin_ids, ctx_seq, past_action_ids, emb_table, W_ih, W_hh, b_ih, b_hh):
    info = plsc.get_sparse_core_info()
    B, STEPS, H = ctx_seq.shape
    plug = plugin_ids.astype(jnp.int32)
    past = past_action_ids.astype(jnp.int32)
    plug_emb, past_mean = _sc_embed(plug, past, emb_table,
                                    info.num_cores, info.num_subcores)
    ctx2d = ctx_seq.reshape(B, STEPS * H)
    h_t = _gru(ctx2d, W_ih, W_hh,
               b_ih.reshape(-1, 1), b_hh.reshape(-1, 1), STEPS, H)
    return jnp.concatenate([plug_emb, h_t.T, past_mean], axis=-1)
